# Initial kernel scaffold; baseline (speedup 1.0000x reference)
import jax
import jax.numpy as jnp
from jax.experimental import pallas as pl


def kernel(x, edge_index, W, b):
    raise NotImplementedError("write your pallas kernel here")


# speedup vs baseline: 17.8131x; 17.8131x over previous
import functools

import jax
import jax.numpy as jnp
from jax import lax
from jax.experimental import pallas as pl
from jax.experimental.pallas import tpu as pltpu
from jax.experimental.pallas import tpu_sc as plsc

N = 10000
FIN = 128
FOUT = 40
NPAD = 10112
FPAD = 48
DUMMY = 10008
NCORES = 2
NSUB = 16
NTILES = NCORES * NSUB
CHUNK = 128
EPT = 10240
NCHUNKS = EPT // CHUNK
EPAD = NTILES * EPT
ROWS_PER_TILE = NPAD // NSUB
DEGW = 16

_MESH = plsc.VectorSubcoreMesh(
    core_axis_name="c", subcore_axis_name="s",
    num_cores=NCORES, num_subcores=NSUB)

_SC_PARAMS = pltpu.CompilerParams(use_tc_tiling_on_sc=False)


def _deg_body(dst_hbm, ones_hbm, z_hbm, out_hbm, dstv, ones_v, acc):
    cid = lax.axis_index("c")
    sid = lax.axis_index("s")
    gwid = cid * NSUB + sid
    pltpu.sync_copy(dst_hbm.at[gwid], dstv)
    pltpu.sync_copy(ones_hbm, ones_v)
    r0 = sid * ROWS_PER_TILE
    pltpu.sync_copy(z_hbm.at[pl.ds(r0, ROWS_PER_TILE)],
                    acc.at[pl.ds(r0, ROWS_PER_TILE)])
    plsc.subcore_barrier()

    def body(j, carry):
        pltpu.sync_copy(ones_v, acc.at[dstv.at[j]], add=True)
        return carry

    lax.fori_loop(0, NCHUNKS, body, 0)
    plsc.subcore_barrier()
    pltpu.sync_copy(acc.at[pl.ds(r0, ROWS_PER_TILE)],
                    out_hbm.at[cid, pl.ds(r0, ROWS_PER_TILE)])


_deg_call = functools.partial(
    pl.kernel,
    out_type=jax.ShapeDtypeStruct((NCORES, NPAD, DEGW), jnp.float32),
    mesh=_MESH,
    scratch_types=[
        pltpu.VMEM((NCHUNKS, CHUNK), jnp.int32),
        pltpu.VMEM((CHUNK, DEGW), jnp.float32),
        pltpu.VMEM_SHARED((NPAD, DEGW), jnp.float32),
    ],
    compiler_params=_SC_PARAMS,
)(_deg_body)


def _hop_body(src_hbm, dst_hbm, u_hbm, z_hbm, out_hbm,
              srcv, dstv, buf0, buf1, acc, sem0, sem1):
    cid = lax.axis_index("c")
    sid = lax.axis_index("s")
    gwid = cid * NSUB + sid
    pltpu.sync_copy(src_hbm.at[gwid], srcv)
    pltpu.sync_copy(dst_hbm.at[gwid], dstv)
    r0 = sid * ROWS_PER_TILE
    pltpu.sync_copy(z_hbm.at[pl.ds(r0, ROWS_PER_TILE)],
                    acc.at[pl.ds(r0, ROWS_PER_TILE)])
    plsc.subcore_barrier()

    def body(j, carry):
        pltpu.sync_copy(u_hbm.at[srcv.at[j]], buf0)
        pltpu.sync_copy(buf0, acc.at[dstv.at[j]], add=True)
        return carry

    lax.fori_loop(0, NCHUNKS, body, 0)
    plsc.subcore_barrier()
    pltpu.sync_copy(acc.at[pl.ds(r0, ROWS_PER_TILE)],
                    out_hbm.at[cid, pl.ds(r0, ROWS_PER_TILE)])


_hop_call = functools.partial(
    pl.kernel,
    out_type=jax.ShapeDtypeStruct((NCORES, NPAD, FPAD), jnp.float32),
    mesh=_MESH,
    scratch_types=[
        pltpu.VMEM((NCHUNKS, CHUNK), jnp.int32),
        pltpu.VMEM((NCHUNKS, CHUNK), jnp.int32),
        pltpu.VMEM((CHUNK, FPAD), jnp.float32),
        pltpu.VMEM((CHUNK, FPAD), jnp.float32),
        pltpu.VMEM_SHARED((NPAD, FPAD), jnp.float32),
        pltpu.SemaphoreType.DMA,
        pltpu.SemaphoreType.DMA,
    ],
    compiler_params=_SC_PARAMS,
)(_hop_body)


def _deg_from_partials(degp_ref):
    deg = (jnp.sum(degp_ref[0], axis=1, keepdims=True)
           + jnp.sum(degp_ref[1], axis=1, keepdims=True) + 1.0)
    return deg


def _prep_body(x_ref, w_ref, degp_ref, u0_ref):
    xw = jnp.dot(x_ref[...], w_ref[...], preferred_element_type=jnp.float32)
    u0_ref[...] = xw * lax.rsqrt(_deg_from_partials(degp_ref))


def _mid_body(degp_ref, s_ref, u_ref, o_ref):
    o_ref[...] = (s_ref[0] + s_ref[1] + u_ref[...]) / _deg_from_partials(degp_ref)


def _fin_body(degp_ref, s_ref, u_ref, b_ref, o_ref):
    o_ref[...] = (lax.rsqrt(_deg_from_partials(degp_ref))
                  * (s_ref[0] + s_ref[1] + u_ref[...]) + b_ref[...])


def kernel(x, edge_index, W, b):
    src = edge_index[0].astype(jnp.int32)
    dst = edge_index[1].astype(jnp.int32)
    pad_e = EPAD - src.shape[0]
    fill = jnp.full((pad_e,), DUMMY, jnp.int32)
    src_p = jnp.concatenate([src, fill]).reshape(NTILES, NCHUNKS, CHUNK)
    dst_p = jnp.concatenate([dst, fill]).reshape(NTILES, NCHUNKS, CHUNK)

    x_p = jnp.pad(x, ((0, NPAD - N), (0, 0)))
    w_p = jnp.pad(W, ((0, 0), (0, FPAD - FOUT)))
    b_p = jnp.pad(b, (0, FPAD - FOUT)).reshape(1, FPAD)
    z48 = jnp.zeros((NPAD, FPAD), jnp.float32)
    z16 = jnp.zeros((NPAD, DEGW), jnp.float32)
    onehot = jnp.zeros((CHUNK, DEGW), jnp.float32).at[:, 0].set(1.0)

    degp = _deg_call(dst_p, onehot, z16)

    u0 = pl.pallas_call(
        _prep_body,
        out_shape=jax.ShapeDtypeStruct((NPAD, FPAD), jnp.float32),
    )(x_p, w_p, degp)

    s1 = _hop_call(src_p, dst_p, u0, z48)

    u1 = pl.pallas_call(
        _mid_body,
        out_shape=jax.ShapeDtypeStruct((NPAD, FPAD), jnp.float32),
    )(degp, s1, u0)

    s2 = _hop_call(src_p, dst_p, u1, z48)

    outp = pl.pallas_call(
        _fin_body,
        out_shape=jax.ShapeDtypeStruct((NPAD, FPAD), jnp.float32),
    )(degp, s2, u1, b_p)

    return outp[:N, :FOUT]


# speedup vs baseline: 37.7231x; 2.1177x over previous
import functools

import jax
import jax.numpy as jnp
from jax import lax
from jax.experimental import pallas as pl
from jax.experimental.pallas import tpu as pltpu
from jax.experimental.pallas import tpu_sc as plsc

N = 10000
FIN = 128
FOUT = 40
NPAD = 10112
FPAD = 48
DUMMY = 10008
NCORES = 2
NSUB = 16
NTILES = NCORES * NSUB
CHUNK = 128
EPT = 10240
NCHUNKS = EPT // CHUNK
EPAD = NTILES * EPT
ROWS_PER_TILE = NPAD // NSUB
DEGW = 16

_MESH = plsc.VectorSubcoreMesh(
    core_axis_name="c", subcore_axis_name="s",
    num_cores=NCORES, num_subcores=NSUB)

_SC_PARAMS = pltpu.CompilerParams(use_tc_tiling_on_sc=False)


def _deg_body(dst_hbm, ones_hbm, z_hbm, out_hbm, dstv, ones_v, acc):
    cid = lax.axis_index("c")
    sid = lax.axis_index("s")
    gwid = cid * NSUB + sid
    pltpu.sync_copy(dst_hbm.at[gwid], dstv)
    pltpu.sync_copy(ones_hbm, ones_v)
    r0 = sid * ROWS_PER_TILE
    pltpu.sync_copy(z_hbm.at[pl.ds(r0, ROWS_PER_TILE)],
                    acc.at[pl.ds(r0, ROWS_PER_TILE)])
    plsc.subcore_barrier()

    def body(j, carry):
        pltpu.sync_copy(ones_v, acc.at[dstv.at[j]], add=True)
        return carry

    lax.fori_loop(0, NCHUNKS, body, 0)
    plsc.subcore_barrier()
    pltpu.sync_copy(acc.at[pl.ds(r0, ROWS_PER_TILE)],
                    out_hbm.at[cid, pl.ds(r0, ROWS_PER_TILE)])


_deg_call = functools.partial(
    pl.kernel,
    out_type=jax.ShapeDtypeStruct((NCORES, NPAD, DEGW), jnp.float32),
    mesh=_MESH,
    scratch_types=[
        pltpu.VMEM((NCHUNKS, CHUNK), jnp.int32),
        pltpu.VMEM((CHUNK, DEGW), jnp.float32),
        pltpu.VMEM_SHARED((NPAD, DEGW), jnp.float32),
    ],
    compiler_params=_SC_PARAMS,
)(_deg_body)


NBUF = 8
LAG = 4


def _hop_body(src_hbm, dst_hbm, u_hbm, z_hbm, out_hbm,
              srcv, dstv, bufs, acc, gsems, ssems):
    cid = lax.axis_index("c")
    sid = lax.axis_index("s")
    gwid = cid * NSUB + sid
    pltpu.sync_copy(src_hbm.at[gwid], srcv)
    pltpu.sync_copy(dst_hbm.at[gwid], dstv)
    r0 = sid * ROWS_PER_TILE
    pltpu.sync_copy(z_hbm.at[pl.ds(r0, ROWS_PER_TILE)],
                    acc.at[pl.ds(r0, ROWS_PER_TILE)])
    plsc.subcore_barrier()

    for b in range(NBUF):
        pltpu.async_copy(u_hbm.at[srcv.at[b]], bufs[b], gsems[b])

    def body(i, carry):
        for b in range(NBUF):
            j = NBUF * i + b
            pltpu.make_async_copy(u_hbm.at[srcv.at[j]], bufs[b], gsems[b]).wait()
            pltpu.async_copy(bufs[b], acc.at[dstv.at[j]], ssems[b], add=True)

            @pl.when((j >= LAG) & (j + LAG < NCHUNKS))
            def _advance():
                bl = (b + NBUF - LAG) % NBUF
                pltpu.make_async_copy(bufs[bl], acc.at[dstv.at[j - LAG]],
                                      ssems[bl]).wait()
                pltpu.async_copy(u_hbm.at[srcv.at[j + LAG]], bufs[bl],
                                 gsems[bl])
        return carry

    lax.fori_loop(0, NCHUNKS // NBUF, body, 0)
    for k in range(NCHUNKS - 2 * LAG, NCHUNKS):
        pltpu.make_async_copy(bufs[k % NBUF], acc.at[dstv.at[k]],
                              ssems[k % NBUF]).wait()
    plsc.subcore_barrier()
    pltpu.sync_copy(acc.at[pl.ds(r0, ROWS_PER_TILE)],
                    out_hbm.at[cid, pl.ds(r0, ROWS_PER_TILE)])


_hop_call = functools.partial(
    pl.kernel,
    out_type=jax.ShapeDtypeStruct((NCORES, NPAD, FPAD), jnp.float32),
    mesh=_MESH,
    scratch_types=[
        pltpu.VMEM((NCHUNKS, CHUNK), jnp.int32),
        pltpu.VMEM((NCHUNKS, CHUNK), jnp.int32),
        [pltpu.VMEM((CHUNK, FPAD), jnp.float32)] * NBUF,
        pltpu.VMEM_SHARED((NPAD, FPAD), jnp.float32),
        [pltpu.SemaphoreType.DMA] * NBUF,
        [pltpu.SemaphoreType.DMA] * NBUF,
    ],
    compiler_params=_SC_PARAMS,
)(_hop_body)


def _deg_from_partials(degp_ref):
    deg = (jnp.sum(degp_ref[0], axis=1, keepdims=True)
           + jnp.sum(degp_ref[1], axis=1, keepdims=True) + 1.0)
    return deg


def _prep_body(x_ref, w_ref, degp_ref, u0_ref):
    xw = jnp.dot(x_ref[...], w_ref[...], preferred_element_type=jnp.float32)
    u0_ref[...] = xw * lax.rsqrt(_deg_from_partials(degp_ref))


def _mid_body(degp_ref, s_ref, u_ref, o_ref):
    o_ref[...] = (s_ref[0] + s_ref[1] + u_ref[...]) / _deg_from_partials(degp_ref)


def _fin_body(degp_ref, s_ref, u_ref, b_ref, o_ref):
    o_ref[...] = (lax.rsqrt(_deg_from_partials(degp_ref))
                  * (s_ref[0] + s_ref[1] + u_ref[...]) + b_ref[...])


def kernel(x, edge_index, W, b):
    src = edge_index[0].astype(jnp.int32)
    dst = edge_index[1].astype(jnp.int32)
    pad_e = EPAD - src.shape[0]
    fill = DUMMY + (jnp.arange(pad_e, dtype=jnp.int32) % 16)
    src_p = jnp.concatenate([src, fill]).reshape(NTILES, NCHUNKS, CHUNK)
    dst_p = jnp.concatenate([dst, fill]).reshape(NTILES, NCHUNKS, CHUNK)

    x_p = jnp.pad(x, ((0, NPAD - N), (0, 0)))
    w_p = jnp.pad(W, ((0, 0), (0, FPAD - FOUT)))
    b_p = jnp.pad(b, (0, FPAD - FOUT)).reshape(1, FPAD)
    z48 = jnp.zeros((NPAD, FPAD), jnp.float32)
    z16 = jnp.zeros((NPAD, DEGW), jnp.float32)
    onehot = jnp.zeros((CHUNK, DEGW), jnp.float32).at[:, 0].set(1.0)

    degp = _deg_call(dst_p, onehot, z16)

    u0 = pl.pallas_call(
        _prep_body,
        out_shape=jax.ShapeDtypeStruct((NPAD, FPAD), jnp.float32),
    )(x_p, w_p, degp)

    s1 = _hop_call(src_p, dst_p, u0, z48)

    u1 = pl.pallas_call(
        _mid_body,
        out_shape=jax.ShapeDtypeStruct((NPAD, FPAD), jnp.float32),
    )(degp, s1, u0)

    s2 = _hop_call(src_p, dst_p, u1, z48)

    outp = pl.pallas_call(
        _fin_body,
        out_shape=jax.ShapeDtypeStruct((NPAD, FPAD), jnp.float32),
    )(degp, s2, u1, b_p)

    return outp[:N, :FOUT]


# speedup vs baseline: 37.9579x; 1.0062x over previous
import functools

import jax
import jax.numpy as jnp
from jax import lax
from jax.experimental import pallas as pl
from jax.experimental.pallas import tpu as pltpu
from jax.experimental.pallas import tpu_sc as plsc

N = 10000
FIN = 128
FOUT = 40
NPAD = 10112
FPAD = 48
DUMMY = 10008
NCORES = 2
NSUB = 16
NTILES = NCORES * NSUB
CHUNK = 256
EPT = 10240
NCHUNKS = EPT // CHUNK
EPAD = NTILES * EPT
ROWS_PER_TILE = NPAD // NSUB
DEGW = 16

_MESH = plsc.VectorSubcoreMesh(
    core_axis_name="c", subcore_axis_name="s",
    num_cores=NCORES, num_subcores=NSUB)

_SC_PARAMS = pltpu.CompilerParams(use_tc_tiling_on_sc=False)


def _deg_body(dst_hbm, ones_hbm, z_hbm, out_hbm, dstv, ones_v, acc):
    cid = lax.axis_index("c")
    sid = lax.axis_index("s")
    gwid = cid * NSUB + sid
    pltpu.sync_copy(dst_hbm.at[gwid], dstv)
    pltpu.sync_copy(ones_hbm, ones_v)
    r0 = sid * ROWS_PER_TILE
    pltpu.sync_copy(z_hbm.at[pl.ds(r0, ROWS_PER_TILE)],
                    acc.at[pl.ds(r0, ROWS_PER_TILE)])
    plsc.subcore_barrier()

    def body(j, carry):
        pltpu.sync_copy(ones_v, acc.at[dstv.at[j]], add=True)
        return carry

    lax.fori_loop(0, NCHUNKS, body, 0)
    plsc.subcore_barrier()
    pltpu.sync_copy(acc.at[pl.ds(r0, ROWS_PER_TILE)],
                    out_hbm.at[cid, pl.ds(r0, ROWS_PER_TILE)])


_deg_call = functools.partial(
    pl.kernel,
    out_type=jax.ShapeDtypeStruct((NCORES, NPAD, DEGW), jnp.float32),
    mesh=_MESH,
    scratch_types=[
        pltpu.VMEM((NCHUNKS, CHUNK), jnp.int32),
        pltpu.VMEM((CHUNK, DEGW), jnp.float32),
        pltpu.VMEM_SHARED((NPAD, DEGW), jnp.float32),
    ],
    compiler_params=_SC_PARAMS,
)(_deg_body)


NBUF = 4
LAG = 2


def _hop_body(src_hbm, dst_hbm, u_hbm, z_hbm, out_hbm,
              srcv, dstv, bufs, acc, gsems, ssems):
    cid = lax.axis_index("c")
    sid = lax.axis_index("s")
    gwid = cid * NSUB + sid
    pltpu.sync_copy(src_hbm.at[gwid], srcv)
    pltpu.sync_copy(dst_hbm.at[gwid], dstv)
    r0 = sid * ROWS_PER_TILE
    pltpu.sync_copy(z_hbm.at[pl.ds(r0, ROWS_PER_TILE)],
                    acc.at[pl.ds(r0, ROWS_PER_TILE)])
    plsc.subcore_barrier()

    for b in range(NBUF):
        pltpu.async_copy(u_hbm.at[srcv.at[b]], bufs[b], gsems[b])

    def body(i, carry):
        for b in range(NBUF):
            j = NBUF * i + b
            pltpu.make_async_copy(u_hbm.at[srcv.at[j]], bufs[b], gsems[b]).wait()
            pltpu.async_copy(bufs[b], acc.at[dstv.at[j]], ssems[b], add=True)

            @pl.when((j >= LAG) & (j + LAG < NCHUNKS))
            def _advance():
                bl = (b + NBUF - LAG) % NBUF
                pltpu.make_async_copy(bufs[bl], acc.at[dstv.at[j - LAG]],
                                      ssems[bl]).wait()
                pltpu.async_copy(u_hbm.at[srcv.at[j + LAG]], bufs[bl],
                                 gsems[bl])
        return carry

    lax.fori_loop(0, NCHUNKS // NBUF, body, 0)
    for k in range(NCHUNKS - 2 * LAG, NCHUNKS):
        pltpu.make_async_copy(bufs[k % NBUF], acc.at[dstv.at[k]],
                              ssems[k % NBUF]).wait()
    plsc.subcore_barrier()
    pltpu.sync_copy(acc.at[pl.ds(r0, ROWS_PER_TILE)],
                    out_hbm.at[cid, pl.ds(r0, ROWS_PER_TILE)])


_hop_call = functools.partial(
    pl.kernel,
    out_type=jax.ShapeDtypeStruct((NCORES, NPAD, FPAD), jnp.float32),
    mesh=_MESH,
    scratch_types=[
        pltpu.VMEM((NCHUNKS, CHUNK), jnp.int32),
        pltpu.VMEM((NCHUNKS, CHUNK), jnp.int32),
        [pltpu.VMEM((CHUNK, FPAD), jnp.float32)] * NBUF,
        pltpu.VMEM_SHARED((NPAD, FPAD), jnp.float32),
        [pltpu.SemaphoreType.DMA] * NBUF,
        [pltpu.SemaphoreType.DMA] * NBUF,
    ],
    compiler_params=_SC_PARAMS,
)(_hop_body)


def _deg_from_partials(degp_ref):
    deg = (jnp.sum(degp_ref[0], axis=1, keepdims=True)
           + jnp.sum(degp_ref[1], axis=1, keepdims=True) + 1.0)
    return deg


def _prep_body(x_ref, w_ref, degp_ref, u0_ref):
    xw = jnp.dot(x_ref[...], w_ref[...], preferred_element_type=jnp.float32)
    u0_ref[...] = xw * lax.rsqrt(_deg_from_partials(degp_ref))


def _mid_body(degp_ref, s_ref, u_ref, o_ref):
    o_ref[...] = (s_ref[0] + s_ref[1] + u_ref[...]) / _deg_from_partials(degp_ref)


def _fin_body(degp_ref, s_ref, u_ref, b_ref, o_ref):
    o_ref[...] = (lax.rsqrt(_deg_from_partials(degp_ref))
                  * (s_ref[0] + s_ref[1] + u_ref[...]) + b_ref[...])


def kernel(x, edge_index, W, b):
    src = edge_index[0].astype(jnp.int32)
    dst = edge_index[1].astype(jnp.int32)
    pad_e = EPAD - src.shape[0]
    fill = DUMMY + (jnp.arange(pad_e, dtype=jnp.int32) % 16)
    src_p = jnp.concatenate([src, fill]).reshape(NTILES, NCHUNKS, CHUNK)
    dst_p = jnp.concatenate([dst, fill]).reshape(NTILES, NCHUNKS, CHUNK)

    x_p = jnp.pad(x, ((0, NPAD - N), (0, 0)))
    w_p = jnp.pad(W, ((0, 0), (0, FPAD - FOUT)))
    b_p = jnp.pad(b, (0, FPAD - FOUT)).reshape(1, FPAD)
    z48 = jnp.zeros((NPAD, FPAD), jnp.float32)
    z16 = jnp.zeros((NPAD, DEGW), jnp.float32)
    onehot = jnp.zeros((CHUNK, DEGW), jnp.float32).at[:, 0].set(1.0)

    degp = _deg_call(dst_p, onehot, z16)

    u0 = pl.pallas_call(
        _prep_body,
        out_shape=jax.ShapeDtypeStruct((NPAD, FPAD), jnp.float32),
    )(x_p, w_p, degp)

    s1 = _hop_call(src_p, dst_p, u0, z48)

    u1 = pl.pallas_call(
        _mid_body,
        out_shape=jax.ShapeDtypeStruct((NPAD, FPAD), jnp.float32),
    )(degp, s1, u0)

    s2 = _hop_call(src_p, dst_p, u1, z48)

    outp = pl.pallas_call(
        _fin_body,
        out_shape=jax.ShapeDtypeStruct((NPAD, FPAD), jnp.float32),
    )(degp, s2, u1, b_p)

    return outp[:N, :FOUT]


# speedup vs baseline: 38.2930x; 1.0088x over previous
import functools

import jax
import jax.numpy as jnp
from jax import lax
from jax.experimental import pallas as pl
from jax.experimental.pallas import tpu as pltpu
from jax.experimental.pallas import tpu_sc as plsc

N = 10000
FIN = 128
FOUT = 40
NPAD = 10112
FPAD = 48
DUMMY = 10008
NCORES = 2
NSUB = 16
NTILES = NCORES * NSUB
CHUNK = 128
EPT = 10240
NCHUNKS = EPT // CHUNK
EPAD = NTILES * EPT
ROWS_PER_TILE = NPAD // NSUB
MERGE_RC = (312, 320)
DEGW = 16

_MESH = plsc.VectorSubcoreMesh(
    core_axis_name="c", subcore_axis_name="s",
    num_cores=NCORES, num_subcores=NSUB)

_SC_PARAMS = pltpu.CompilerParams(use_tc_tiling_on_sc=False)

NBUF = 4
LAG = 2


def _edge_pipeline(srcv, dstv, u_sh, acc, bufs, gsems, ssems):
    for b in range(NBUF):
        pltpu.async_copy(u_sh.at[srcv.at[b]], bufs[b], gsems[b])

    def body(i, carry):
        for b in range(NBUF):
            j = NBUF * i + b
            pltpu.make_async_copy(u_sh.at[srcv.at[j]], bufs[b], gsems[b]).wait()
            pltpu.async_copy(bufs[b], acc.at[dstv.at[j]], ssems[b], add=True)

            @pl.when((j >= LAG) & (j + LAG < NCHUNKS))
            def _advance():
                bl = (b + NBUF - LAG) % NBUF
                pltpu.make_async_copy(bufs[bl], acc.at[dstv.at[j - LAG]],
                                      ssems[bl]).wait()
                pltpu.async_copy(u_sh.at[srcv.at[j + LAG]], bufs[bl],
                                 gsems[bl])
        return carry

    lax.fori_loop(0, NCHUNKS // NBUF, body, 0)
    for k in range(NCHUNKS - 2 * LAG, NCHUNKS):
        pltpu.make_async_copy(bufs[k % NBUF], acc.at[dstv.at[k]],
                              ssems[k % NBUF]).wait()


def _deg_body(dst_hbm, ones_hbm, z_hbm, out_hbm, dstv, ones_v, acc):
    cid = lax.axis_index("c")
    sid = lax.axis_index("s")
    gwid = cid * NSUB + sid
    pltpu.sync_copy(dst_hbm.at[gwid], dstv)
    pltpu.sync_copy(ones_hbm, ones_v)
    r0 = sid * ROWS_PER_TILE
    pltpu.sync_copy(z_hbm.at[pl.ds(r0, ROWS_PER_TILE)],
                    acc.at[pl.ds(r0, ROWS_PER_TILE)])
    plsc.subcore_barrier()

    def body(j, carry):
        pltpu.sync_copy(ones_v, acc.at[dstv.at[j]], add=True)
        return carry

    lax.fori_loop(0, NCHUNKS, body, 0)
    plsc.subcore_barrier()
    pltpu.sync_copy(acc.at[pl.ds(r0, ROWS_PER_TILE)],
                    out_hbm.at[cid, pl.ds(r0, ROWS_PER_TILE)])


_deg_call = functools.partial(
    pl.kernel,
    out_type=jax.ShapeDtypeStruct((NCORES, NPAD, DEGW), jnp.float32),
    mesh=_MESH,
    scratch_types=[
        pltpu.VMEM((NCHUNKS, CHUNK), jnp.int32),
        pltpu.VMEM((CHUNK, DEGW), jnp.float32),
        pltpu.VMEM_SHARED((NPAD, DEGW), jnp.float32),
    ],
    compiler_params=_SC_PARAMS,
)(_deg_body)


def _hop1_body(src_hbm, dst_hbm, u_hbm, z_hbm, out_hbm,
               srcv, dstv, bufs, acc, gsems, ssems):
    cid = lax.axis_index("c")
    sid = lax.axis_index("s")
    gwid = cid * NSUB + sid
    pltpu.sync_copy(src_hbm.at[gwid], srcv)
    pltpu.sync_copy(dst_hbm.at[gwid], dstv)
    r0 = sid * ROWS_PER_TILE
    pltpu.sync_copy(z_hbm.at[pl.ds(r0, ROWS_PER_TILE)],
                    acc.at[pl.ds(r0, ROWS_PER_TILE)])
    plsc.subcore_barrier()
    _edge_pipeline(srcv, dstv, u_hbm, acc, bufs, gsems, ssems)
    plsc.subcore_barrier()
    pltpu.sync_copy(acc.at[pl.ds(r0, ROWS_PER_TILE)],
                    out_hbm.at[cid, pl.ds(r0, ROWS_PER_TILE)])


_hop1_call = functools.partial(
    pl.kernel,
    out_type=jax.ShapeDtypeStruct((NCORES, NPAD, FPAD), jnp.float32),
    mesh=_MESH,
    scratch_types=[
        pltpu.VMEM((NCHUNKS, CHUNK), jnp.int32),
        pltpu.VMEM((NCHUNKS, CHUNK), jnp.int32),
        [pltpu.VMEM((CHUNK, FPAD), jnp.float32)] * NBUF,
        pltpu.VMEM_SHARED((NPAD, FPAD), jnp.float32),
        [pltpu.SemaphoreType.DMA] * NBUF,
        [pltpu.SemaphoreType.DMA] * NBUF,
    ],
    compiler_params=_SC_PARAMS,
)(_hop1_body)


def _hop2_body(src_hbm, dst_hbm, s1_hbm, u0_hbm, scl_hbm, z_hbm,
               out_hbm, u1_hbm,
               srcv, dstv, bufs, ma, mb, mc, ms, acc, gsems, ssems):
    cid = lax.axis_index("c")
    sid = lax.axis_index("s")
    gwid = cid * NSUB + sid
    pltpu.sync_copy(src_hbm.at[gwid], srcv)
    pltpu.sync_copy(dst_hbm.at[gwid], dstv)
    r0 = sid * ROWS_PER_TILE
    pltpu.sync_copy(z_hbm.at[pl.ds(r0, ROWS_PER_TILE)],
                    acc.at[pl.ds(r0, ROWS_PER_TILE)])

    off = 0
    for rc in MERGE_RC:
        base = r0 + off
        pltpu.sync_copy(s1_hbm.at[0, pl.ds(base, rc)], ma.at[pl.ds(0, rc)])
        pltpu.sync_copy(s1_hbm.at[1, pl.ds(base, rc)], mb.at[pl.ds(0, rc)])
        pltpu.sync_copy(u0_hbm.at[pl.ds(base, rc)], mc.at[pl.ds(0, rc)])
        pltpu.sync_copy(scl_hbm.at[pl.ds(base, rc)], ms.at[pl.ds(0, rc)])

        def mrow(r, carry):
            sv = ms[r, :]
            for k in range(FPAD // 16):
                col = pl.ds(16 * k, 16)
                ma[r, col] = (ma[r, col] + mb[r, col] + mc[r, col]) * sv
            return carry

        lax.fori_loop(0, rc, mrow, 0)
        pltpu.sync_copy(ma.at[pl.ds(0, rc)], u1_hbm.at[pl.ds(base, rc)])
        off += rc

    plsc.subcore_barrier()
    _edge_pipeline(srcv, dstv, u1_hbm, acc, bufs, gsems, ssems)
    plsc.subcore_barrier()
    pltpu.sync_copy(acc.at[pl.ds(r0, ROWS_PER_TILE)],
                    out_hbm.at[cid, pl.ds(r0, ROWS_PER_TILE)])


_hop2_call = functools.partial(
    pl.kernel,
    out_type=(jax.ShapeDtypeStruct((NCORES, NPAD, FPAD), jnp.float32),
              jax.ShapeDtypeStruct((NPAD, FPAD), jnp.float32)),
    mesh=_MESH,
    scratch_types=[
        pltpu.VMEM((NCHUNKS, CHUNK), jnp.int32),
        pltpu.VMEM((NCHUNKS, CHUNK), jnp.int32),
        [pltpu.VMEM((CHUNK, FPAD), jnp.float32)] * NBUF,
        pltpu.VMEM((max(MERGE_RC), FPAD), jnp.float32),
        pltpu.VMEM((max(MERGE_RC), FPAD), jnp.float32),
        pltpu.VMEM((max(MERGE_RC), FPAD), jnp.float32),
        pltpu.VMEM((max(MERGE_RC), DEGW), jnp.float32),
        pltpu.VMEM_SHARED((NPAD, FPAD), jnp.float32),
        [pltpu.SemaphoreType.DMA] * NBUF,
        [pltpu.SemaphoreType.DMA] * NBUF,
    ],
    compiler_params=_SC_PARAMS,
)(_hop2_body)


def _deg_from_partials(degp_ref):
    deg = (jnp.sum(degp_ref[0], axis=1, keepdims=True)
           + jnp.sum(degp_ref[1], axis=1, keepdims=True) + 1.0)
    return deg


def _prep_body(x_ref, w_ref, degp_ref, u0_ref, scl_ref):
    deg = _deg_from_partials(degp_ref)
    xw = jnp.dot(x_ref[...], w_ref[...], preferred_element_type=jnp.float32)
    u0_ref[...] = xw * lax.rsqrt(deg)
    scl_ref[...] = jnp.broadcast_to(1.0 / deg, (NPAD, DEGW))


def _fin_body(degp_ref, s_ref, u_ref, b_ref, o_ref):
    o_ref[...] = (lax.rsqrt(_deg_from_partials(degp_ref))
                  * (s_ref[0] + s_ref[1] + u_ref[...]) + b_ref[...])


def kernel(x, edge_index, W, b):
    src = edge_index[0].astype(jnp.int32)
    dst = edge_index[1].astype(jnp.int32)
    pad_e = EPAD - src.shape[0]
    fill = DUMMY + (jnp.arange(pad_e, dtype=jnp.int32) % 16)
    src_p = jnp.concatenate([src, fill]).reshape(NTILES, NCHUNKS, CHUNK)
    dst_p = jnp.concatenate([dst, fill]).reshape(NTILES, NCHUNKS, CHUNK)

    x_p = jnp.pad(x, ((0, NPAD - N), (0, 0)))
    w_p = jnp.pad(W, ((0, 0), (0, FPAD - FOUT)))
    b_p = jnp.pad(b, (0, FPAD - FOUT)).reshape(1, FPAD)
    z48 = jnp.zeros((NPAD, FPAD), jnp.float32)
    z16 = jnp.zeros((NPAD, DEGW), jnp.float32)
    onehot = jnp.zeros((CHUNK, DEGW), jnp.float32).at[:, 0].set(1.0)

    degp = _deg_call(dst_p, onehot, z16)

    u0, scl = pl.pallas_call(
        _prep_body,
        out_shape=(jax.ShapeDtypeStruct((NPAD, FPAD), jnp.float32),
                   jax.ShapeDtypeStruct((NPAD, DEGW), jnp.float32)),
    )(x_p, w_p, degp)

    s1 = _hop1_call(src_p, dst_p, u0, z48)

    s2, u1 = _hop2_call(src_p, dst_p, s1, u0, scl, z48)

    outp = pl.pallas_call(
        _fin_body,
        out_shape=jax.ShapeDtypeStruct((NPAD, FPAD), jnp.float32),
    )(degp, s2, u1, b_p)

    return outp[:N, :FOUT]


# speedup vs baseline: 39.2015x; 1.0237x over previous
import functools

import jax
import jax.numpy as jnp
from jax import lax
from jax.experimental import pallas as pl
from jax.experimental.pallas import tpu as pltpu
from jax.experimental.pallas import tpu_sc as plsc

N = 10000
FIN = 128
FOUT = 40
NPAD = 10112
FPAD = 48
DUMMY = 10008
NCORES = 2
NSUB = 16
NTILES = NCORES * NSUB
CHUNK = 128
EPT = 10240
NCHUNKS = EPT // CHUNK
EPAD = NTILES * EPT
ROWS_PER_TILE = NPAD // NSUB
MERGE_RC = (312, 320)
DEGW = 16

_MESH = plsc.VectorSubcoreMesh(
    core_axis_name="c", subcore_axis_name="s",
    num_cores=NCORES, num_subcores=NSUB)

_SC_PARAMS = pltpu.CompilerParams(use_tc_tiling_on_sc=False)

NBUF1 = 8
NBUF2 = 4


def _edge_pipeline(srcv, dstv, u_ref, acc, bufs, gsems, ssems):
    nbuf = len(bufs)
    lag = nbuf // 2
    assert (NCHUNKS - 2 * lag) % nbuf == 0

    def gather(j, b, sl):
        pltpu.async_copy(u_ref.at[sl], bufs[b], gsems[b])

    def wait_gather(j, b, sl):
        pltpu.make_async_copy(u_ref.at[sl], bufs[b], gsems[b]).wait()

    def scatter(j, b, sl):
        pltpu.async_copy(bufs[b], acc.at[sl], ssems[b], add=True)

    def wait_scatter(j, b, sl):
        pltpu.make_async_copy(bufs[b], acc.at[sl], ssems[b]).wait()

    for j in range(nbuf):
        gather(j, j, srcv.at[j])
    for j in range(lag):
        wait_gather(j, j, srcv.at[j])
        scatter(j, j, dstv.at[j])

    def body(i, carry):
        for m in range(nbuf):
            j = nbuf * i + lag + m
            b = (lag + m) % nbuf
            wait_gather(j, b, srcv.at[j])
            scatter(j, b, dstv.at[j])
            wait_scatter(j - lag, m, dstv.at[j - lag])
            gather(j + lag, m, srcv.at[j + lag])
        return carry

    lax.fori_loop(0, (NCHUNKS - 2 * lag) // nbuf, body, 0)
    for j in range(NCHUNKS - lag, NCHUNKS):
        wait_gather(j, j % nbuf, srcv.at[j])
        scatter(j, j % nbuf, dstv.at[j])
    for j in range(NCHUNKS - 2 * lag, NCHUNKS):
        wait_scatter(j, j % nbuf, dstv.at[j])


def _deg_body(dst_hbm, ones_hbm, z_hbm, out_hbm, dstv, ones_v, acc, ssems):
    cid = lax.axis_index("c")
    sid = lax.axis_index("s")
    gwid = cid * NSUB + sid
    pltpu.sync_copy(dst_hbm.at[gwid], dstv)
    pltpu.sync_copy(ones_hbm, ones_v)
    r0 = sid * ROWS_PER_TILE
    pltpu.sync_copy(z_hbm.at[pl.ds(r0, ROWS_PER_TILE)],
                    acc.at[pl.ds(r0, ROWS_PER_TILE)])
    plsc.subcore_barrier()

    nsem = len(ssems)
    for j in range(nsem):
        pltpu.async_copy(ones_v, acc.at[dstv.at[j]], ssems[j], add=True)

    def body(i, carry):
        for m in range(nsem):
            j = nsem * i + nsem + m
            pltpu.make_async_copy(ones_v, acc.at[dstv.at[j - nsem]],
                                  ssems[m]).wait()
            pltpu.async_copy(ones_v, acc.at[dstv.at[j]], ssems[m], add=True)
        return carry

    lax.fori_loop(0, (NCHUNKS - nsem) // nsem, body, 0)
    for j in range(NCHUNKS - nsem, NCHUNKS):
        pltpu.make_async_copy(ones_v, acc.at[dstv.at[j]],
                              ssems[j % nsem]).wait()
    plsc.subcore_barrier()
    pltpu.sync_copy(acc.at[pl.ds(r0, ROWS_PER_TILE)],
                    out_hbm.at[cid, pl.ds(r0, ROWS_PER_TILE)])


_deg_call = functools.partial(
    pl.kernel,
    out_type=jax.ShapeDtypeStruct((NCORES, NPAD, DEGW), jnp.float32),
    mesh=_MESH,
    scratch_types=[
        pltpu.VMEM((NCHUNKS, CHUNK), jnp.int32),
        pltpu.VMEM((CHUNK, DEGW), jnp.float32),
        pltpu.VMEM_SHARED((NPAD, DEGW), jnp.float32),
        [pltpu.SemaphoreType.DMA] * 8,
    ],
    compiler_params=_SC_PARAMS,
)(_deg_body)


def _hop1_body(src_hbm, dst_hbm, u_hbm, z_hbm, out_hbm,
               srcv, dstv, bufs, acc, gsems, ssems):
    cid = lax.axis_index("c")
    sid = lax.axis_index("s")
    gwid = cid * NSUB + sid
    pltpu.sync_copy(src_hbm.at[gwid], srcv)
    pltpu.sync_copy(dst_hbm.at[gwid], dstv)
    r0 = sid * ROWS_PER_TILE
    pltpu.sync_copy(z_hbm.at[pl.ds(r0, ROWS_PER_TILE)],
                    acc.at[pl.ds(r0, ROWS_PER_TILE)])
    plsc.subcore_barrier()
    _edge_pipeline(srcv, dstv, u_hbm, acc, bufs, gsems, ssems)
    plsc.subcore_barrier()
    pltpu.sync_copy(acc.at[pl.ds(r0, ROWS_PER_TILE)],
                    out_hbm.at[cid, pl.ds(r0, ROWS_PER_TILE)])


_hop1_call = functools.partial(
    pl.kernel,
    out_type=jax.ShapeDtypeStruct((NCORES, NPAD, FPAD), jnp.float32),
    mesh=_MESH,
    scratch_types=[
        pltpu.VMEM((NCHUNKS, CHUNK), jnp.int32),
        pltpu.VMEM((NCHUNKS, CHUNK), jnp.int32),
        [pltpu.VMEM((CHUNK, FPAD), jnp.float32)] * NBUF1,
        pltpu.VMEM_SHARED((NPAD, FPAD), jnp.float32),
        [pltpu.SemaphoreType.DMA] * NBUF1,
        [pltpu.SemaphoreType.DMA] * NBUF1,
    ],
    compiler_params=_SC_PARAMS,
)(_hop1_body)


def _hop2_body(src_hbm, dst_hbm, s1_hbm, u0_hbm, scl_hbm, z_hbm,
               out_hbm, u1_hbm,
               srcv, dstv, bufs, ma, mb, mc, ms, acc, gsems, ssems):
    cid = lax.axis_index("c")
    sid = lax.axis_index("s")
    gwid = cid * NSUB + sid
    pltpu.sync_copy(src_hbm.at[gwid], srcv)
    pltpu.sync_copy(dst_hbm.at[gwid], dstv)
    r0 = sid * ROWS_PER_TILE
    pltpu.sync_copy(z_hbm.at[pl.ds(r0, ROWS_PER_TILE)],
                    acc.at[pl.ds(r0, ROWS_PER_TILE)])

    off = 0
    for rc in MERGE_RC:
        base = r0 + off
        pltpu.sync_copy(s1_hbm.at[0, pl.ds(base, rc)], ma.at[pl.ds(0, rc)])
        pltpu.sync_copy(s1_hbm.at[1, pl.ds(base, rc)], mb.at[pl.ds(0, rc)])
        pltpu.sync_copy(u0_hbm.at[pl.ds(base, rc)], mc.at[pl.ds(0, rc)])
        pltpu.sync_copy(scl_hbm.at[pl.ds(base, rc)], ms.at[pl.ds(0, rc)])

        def mrow(r, carry):
            sv = ms[r, :]
            for k in range(FPAD // 16):
                col = pl.ds(16 * k, 16)
                ma[r, col] = (ma[r, col] + mb[r, col] + mc[r, col]) * sv
            return carry

        lax.fori_loop(0, rc, mrow, 0)
        pltpu.sync_copy(ma.at[pl.ds(0, rc)], u1_hbm.at[pl.ds(base, rc)])
        off += rc

    plsc.subcore_barrier()
    _edge_pipeline(srcv, dstv, u1_hbm, acc, bufs, gsems, ssems)
    plsc.subcore_barrier()
    pltpu.sync_copy(acc.at[pl.ds(r0, ROWS_PER_TILE)],
                    out_hbm.at[cid, pl.ds(r0, ROWS_PER_TILE)])


_hop2_call = functools.partial(
    pl.kernel,
    out_type=(jax.ShapeDtypeStruct((NCORES, NPAD, FPAD), jnp.float32),
              jax.ShapeDtypeStruct((NPAD, FPAD), jnp.float32)),
    mesh=_MESH,
    scratch_types=[
        pltpu.VMEM((NCHUNKS, CHUNK), jnp.int32),
        pltpu.VMEM((NCHUNKS, CHUNK), jnp.int32),
        [pltpu.VMEM((CHUNK, FPAD), jnp.float32)] * NBUF2,
        pltpu.VMEM((max(MERGE_RC), FPAD), jnp.float32),
        pltpu.VMEM((max(MERGE_RC), FPAD), jnp.float32),
        pltpu.VMEM((max(MERGE_RC), FPAD), jnp.float32),
        pltpu.VMEM((max(MERGE_RC), DEGW), jnp.float32),
        pltpu.VMEM_SHARED((NPAD, FPAD), jnp.float32),
        [pltpu.SemaphoreType.DMA] * NBUF2,
        [pltpu.SemaphoreType.DMA] * NBUF2,
    ],
    compiler_params=_SC_PARAMS,
)(_hop2_body)


def _deg_from_partials(degp_ref):
    deg = (jnp.sum(degp_ref[0], axis=1, keepdims=True)
           + jnp.sum(degp_ref[1], axis=1, keepdims=True) + 1.0)
    return deg


def _prep_body(x_ref, w_ref, degp_ref, u0_ref, scl_ref):
    deg = _deg_from_partials(degp_ref)
    xw = jnp.dot(x_ref[...], w_ref[...], preferred_element_type=jnp.float32)
    u0_ref[...] = xw * lax.rsqrt(deg)
    scl_ref[...] = jnp.broadcast_to(1.0 / deg, (NPAD, DEGW))


def _fin_body(degp_ref, s_ref, u_ref, b_ref, o_ref):
    o_ref[...] = (lax.rsqrt(_deg_from_partials(degp_ref))
                  * (s_ref[0] + s_ref[1] + u_ref[...]) + b_ref[...])


def kernel(x, edge_index, W, b):
    src = edge_index[0].astype(jnp.int32)
    dst = edge_index[1].astype(jnp.int32)
    pad_e = EPAD - src.shape[0]
    fill = DUMMY + (jnp.arange(pad_e, dtype=jnp.int32) % 16)
    src_p = jnp.concatenate([src, fill]).reshape(NTILES, NCHUNKS, CHUNK)
    dst_p = jnp.concatenate([dst, fill]).reshape(NTILES, NCHUNKS, CHUNK)

    x_p = jnp.pad(x, ((0, NPAD - N), (0, 0)))
    w_p = jnp.pad(W, ((0, 0), (0, FPAD - FOUT)))
    b_p = jnp.pad(b, (0, FPAD - FOUT)).reshape(1, FPAD)
    z48 = jnp.zeros((NPAD, FPAD), jnp.float32)
    z16 = jnp.zeros((NPAD, DEGW), jnp.float32)
    onehot = jnp.zeros((CHUNK, DEGW), jnp.float32).at[:, 0].set(1.0)

    degp = _deg_call(dst_p, onehot, z16)

    u0, scl = pl.pallas_call(
        _prep_body,
        out_shape=(jax.ShapeDtypeStruct((NPAD, FPAD), jnp.float32),
                   jax.ShapeDtypeStruct((NPAD, DEGW), jnp.float32)),
    )(x_p, w_p, degp)

    s1 = _hop1_call(src_p, dst_p, u0, z48)

    s2, u1 = _hop2_call(src_p, dst_p, s1, u0, scl, z48)

    outp = pl.pallas_call(
        _fin_body,
        out_shape=jax.ShapeDtypeStruct((NPAD, FPAD), jnp.float32),
    )(degp, s2, u1, b_p)

    return outp[:N, :FOUT]
